# baseline (speedup 1.0000x reference)
import functools

import jax
import jax.numpy as jnp
from jax import lax
from jax.experimental import pallas as pl
from jax.experimental.pallas import tpu as pltpu
from jax.experimental.pallas import tpu_sc as plsc

_N = 10000
_D = 128
_E = 160000
_S = 16
_H = 1024

_INTERPRET = False

_MESH = plsc.VectorSubcoreMesh(core_axis_name="c", subcore_axis_name="s")
_NC = 2
_NS = 16


def _lrelu(x):
    return jnp.where(x >= 0, x, 0.01 * x)


_A_CH = 256
_NPAD = 10240


_A_ROWS = 5632
_A_HALF = _N // 2


def _gcn1_agg_sc(x, src, dst, zeros_nd):
    per_tile = _E // _NS

    @functools.partial(
        pl.kernel,
        out_type=jax.ShapeDtypeStruct((_NC, _A_ROWS, _D), jnp.float32),
        mesh=_MESH,
        scratch_types=dict(
            acc=pltpu.VMEM_SHARED((_A_ROWS, _D), jnp.float32),
            gidx=pltpu.VMEM((_A_CH,), jnp.int32),
            sidx=pltpu.VMEM((_A_CH,), jnp.int32),
            rows=pltpu.VMEM((_A_CH, _D), jnp.float32),
            gidx_t=pltpu.VMEM((16,), jnp.int32),
            sidx_t=pltpu.VMEM((16,), jnp.int32),
            rows_t=pltpu.VMEM((16, _D), jnp.float32),
            sem=pltpu.SemaphoreType.DMA,
        ),
    )
    def k(x_hbm, src_hbm, dst_hbm, z_nd, a1_out, *, acc,
          gidx, sidx, rows, gidx_t, sidx_t, rows_t, sem):
        core = lax.axis_index("c")
        sid = lax.axis_index("s")
        stripe = _A_ROWS // _NS
        r0 = sid * stripe
        pltpu.sync_copy(z_nd.at[pl.ds(r0, stripe)], acc.at[pl.ds(r0, stripe)])
        plsc.subcore_barrier()
        base = sid * per_tile
        nb = core * _A_HALF

        def clamp(buf, n):
            @pl.loop(0, n // 16)
            def _(i):
                iv = buf[pl.ds(i * 16, 16)]
                lv = iv - nb
                m = (lv >= 0) & (lv < _A_HALF)
                buf[pl.ds(i * 16, 16)] = jnp.where(m, lv, _A_HALF)

        def chunk(off, gref, sref, gb, sb, rb, n):
            pltpu.sync_copy(gref.at[pl.ds(off, n)], gb)
            pltpu.sync_copy(sref.at[pl.ds(off, n)], sb)
            clamp(sb, n)
            pltpu.async_copy(x_hbm.at[gb], rb, sem).wait()
            pltpu.sync_copy(rb, acc.at[sb], add=True)

        nfull = per_tile // _A_CH
        tail = per_tile - nfull * _A_CH

        def direction(gref, sref):
            @pl.loop(0, nfull)
            def _(c):
                chunk(base + c * _A_CH, gref, sref, gidx, sidx, rows, _A_CH)
            if tail:
                chunk(base + nfull * _A_CH, gref, sref, gidx_t, sidx_t,
                      rows_t, tail)

        direction(src_hbm, dst_hbm)
        direction(dst_hbm, src_hbm)

        plsc.subcore_barrier()
        pltpu.sync_copy(acc.at[pl.ds(r0, stripe)],
                        a1_out.at[core, pl.ds(r0, stripe)])

    return k(x, src, dst, zeros_nd)


_C_CH = 128


def _edge_features_sc(h, src, dst, actions):
    per_tile = _E // (_NC * _NS)
    nfull = per_tile // _C_CH
    tail = per_tile - nfull * _C_CH
    w2 = 2 * _D

    @functools.partial(
        pl.kernel,
        out_type=(
            jax.ShapeDtypeStruct((_E, w2), jnp.float32),
            jax.ShapeDtypeStruct((_NC, _NS, 2 * w2), jnp.float32),
        ),
        mesh=_MESH,
        scratch_types=dict(
            hs=pltpu.VMEM((_C_CH, w2), jnp.float32),
            hd=pltpu.VMEM((_C_CH, w2), jnp.float32),
            sidx=pltpu.VMEM((_C_CH,), jnp.int32),
            didx=pltpu.VMEM((_C_CH,), jnp.int32),
            act=pltpu.VMEM((_C_CH,), jnp.float32),
            hs_t=pltpu.VMEM((8, w2), jnp.float32),
            hd_t=pltpu.VMEM((8, w2), jnp.float32),
            sidx_t=pltpu.VMEM((8,), jnp.int32),
            didx_t=pltpu.VMEM((8,), jnp.int32),
            act_t=pltpu.VMEM((16,), jnp.float32),
            stats=pltpu.VMEM((2 * w2,), jnp.float32),
            sem=pltpu.SemaphoreType.DMA,
            sem2=pltpu.SemaphoreType.DMA,
        ),
    )
    def k(h_hbm, src_hbm, dst_hbm, act_hbm, ef_out, st_out, *, hs, hd, sidx,
          didx, act, hs_t, hd_t, sidx_t, didx_t, act_t, stats, sem, sem2):
        core = lax.axis_index("c")
        sid = lax.axis_index("s")
        wid = core * _NS + sid
        base = wid * per_tile

        @pl.loop(0, 2 * w2 // 16)
        def _(i):
            stats[pl.ds(i * 16, 16)] = jnp.zeros((16,), jnp.float32)

        def rows16(hsb, hdb, row0, avec, lanes):
            for l in range(lanes):
                r = row0 + l
                av = avec[l]
                for j in range(w2 // 16):
                    cl = pl.ds(j * 16, 16)
                    v = (hsb[r, cl] + hdb[r, cl]) * av
                    ef = jnp.where(v >= 0, v, 0.01 * v)
                    hsb[r, cl] = ef
                    stats[cl] = stats[cl] + ef
                    qcl = pl.ds(w2 + j * 16, 16)
                    stats[qcl] = stats[qcl] + ef * ef

        def chunk(off, sb, db, ab, hsb, hdb, n):
            pltpu.sync_copy(src_hbm.at[pl.ds(off, n)], sb)
            pltpu.sync_copy(dst_hbm.at[pl.ds(off, n)], db)
            pltpu.sync_copy(act_hbm.at[pl.ds(off, n)], ab.at[pl.ds(0, n)])
            cs = pltpu.async_copy(h_hbm.at[sb], hsb.at[pl.ds(0, n)], sem)
            cd = pltpu.async_copy(h_hbm.at[db], hdb.at[pl.ds(0, n)], sem2)
            cs.wait()
            cd.wait()

            @pl.loop(0, n // 16)
            def _(g):
                avec = ab[pl.ds(g * 16, 16)]
                rows16(hsb, hdb, g * 16, avec, 16)

            rem = n % 16
            if rem:
                avec = ab[pl.ds(0, 16)]
                rows16(hsb, hdb, n - rem, avec, rem)

            pltpu.sync_copy(hsb.at[pl.ds(0, n)], ef_out.at[pl.ds(off, n)])

        @pl.loop(0, nfull)
        def _(c):
            chunk(base + c * _C_CH, sidx, didx, act, hs, hd, _C_CH)
        if tail:
            chunk(base + nfull * _C_CH, sidx_t, didx_t, act_t, hs_t, hd_t,
                  tail)

        pltpu.sync_copy(stats, st_out.at[core, sid])

    return k(h, src, dst, actions)


def _h_body(x_ref, a_ref, deg_ref, w_ref, out_ref):
    deg = jnp.maximum(deg_ref[...], 1.0)
    x2 = x_ref[...] + a_ref[...] / deg
    out_ref[...] = _lrelu(jnp.dot(x2, w_ref[...], preferred_element_type=jnp.float32))


def _h_matmul(x, a1, deg1, w1ab):
    bm = 2000
    grid = (_N // bm,)
    return pl.pallas_call(
        _h_body,
        grid=grid,
        in_specs=[
            pl.BlockSpec((bm, _D), lambda i: (i, 0)),
            pl.BlockSpec((bm, _D), lambda i: (i, 0)),
            pl.BlockSpec((bm, 1), lambda i: (i, 0)),
            pl.BlockSpec((_D, 2 * _D), lambda i: (0, 0)),
        ],
        out_specs=pl.BlockSpec((bm, 2 * _D), lambda i: (i, 0)),
        out_shape=jax.ShapeDtypeStruct((_N, 2 * _D), jnp.float32),
        interpret=_INTERPRET,
    )(x, a1, deg1, w1ab)


def _mlp_body(sgz_ref, agg_ref, deg_ref, a_ref, b_ref, wblk_ref,
              w1a_ref, b1a_ref, w2a_ref, b2a_ref, w3a_ref, b3a_ref,
              w1b_ref, b1b_ref, w2b_ref, b2b_ref, w3b_ref, b3b_ref,
              out_ref):
    degr = deg_ref[...]
    deg = jnp.maximum(degr, 1.0)
    ind = jnp.where(degr > 0, 1.0, 0.0)
    pre = (a_ref[...] * (sgz_ref[...] + agg_ref[...] / deg)
           + b_ref[...] * (1.0 + ind))
    g = jnp.dot(pre, wblk_ref[...], preferred_element_type=jnp.float32)
    lr = _lrelu(_lrelu(g))
    gb = lr.shape[0] // _S
    pooled = jnp.mean(lr.reshape(gb, _S, 2 * _D), axis=1)
    p1 = pooled[:, :_D]
    p2 = pooled[:, _D:]

    def value(p, w1, b1, w2, b2, w3, b3):
        x = _lrelu(jnp.dot(p, w1[...], preferred_element_type=jnp.float32) + b1[...])
        x = _lrelu(jnp.dot(x, w2[...], preferred_element_type=jnp.float32) + b2[...])
        return jnp.dot(x, w3[...], preferred_element_type=jnp.float32) + b3[...]

    q1 = value(p1, w1a_ref, b1a_ref, w2a_ref, b2a_ref, w3a_ref, b3a_ref)
    q2 = value(p2, w1b_ref, b1b_ref, w2b_ref, b2b_ref, w3b_ref, b3b_ref)
    out_ref[...] = jnp.concatenate([q1, q2], axis=-1)


def _mlp(sgz, agg, deg2, a_bn, b_bn, wblk, wt):
    bm = 6400
    gb = bm // _S
    grid = (_E // bm,)
    full = lambda shape: pl.BlockSpec(shape, lambda i: tuple(0 for _ in shape))
    return pl.pallas_call(
        _mlp_body,
        grid=grid,
        in_specs=[
            pl.BlockSpec((bm, 2 * _D), lambda i: (i, 0)),
            pl.BlockSpec((bm, 2 * _D), lambda i: (i, 0)),
            pl.BlockSpec((bm, 1), lambda i: (i, 0)),
            full((1, 2 * _D)), full((1, 2 * _D)),
            full((2 * _D, 2 * _D)),
            full((_D, _H)), full((1, _H)), full((_H, _H)), full((1, _H)),
            full((_H, 1)), full((1, 1)),
            full((_D, _H)), full((1, _H)), full((_H, _H)), full((1, _H)),
            full((_H, 1)), full((1, 1)),
        ],
        out_specs=pl.BlockSpec((gb, 2), lambda i: (i, 0)),
        out_shape=jax.ShapeDtypeStruct((_N, 2), jnp.float32),
        interpret=_INTERPRET,
    )(sgz, agg, deg2, a_bn, b_bn, wblk, *wt)


def kernel(node_features, actions, edge_index, sub_graphs, sep_subgraphs,
           Wg1a, Wg1b, Wg2a, Wg2b, gamma, beta,
           W1a, b1a, W2a, b2a, W3a, b3a,
           W1b, b1b, W2b, b2b, W3b, b3b):
    x = node_features
    src = edge_index[0]
    dst = edge_index[1]
    src2 = jnp.concatenate([src, dst])
    dst2 = jnp.concatenate([dst, src])

    a1p = _gcn1_agg_sc(x, src, dst, jnp.zeros((_A_ROWS, _D), jnp.float32))
    a1 = jnp.concatenate([a1p[0, :_A_HALF], a1p[1, :_A_HALF]], axis=0)
    dst2 = jnp.concatenate([dst, src])
    deg1 = jax.ops.segment_sum(jnp.ones((2 * _E,), jnp.float32), dst2,
                               num_segments=_N)

    w1ab = jnp.concatenate([Wg1a, Wg1b], axis=1)
    h = _h_matmul(x, a1, deg1[:, None], w1ab)

    ef, stp = _edge_features_sc(h, src, dst, actions)
    sums = jnp.sum(stp.reshape(_NC * _NS, 2 * 2 * _D), axis=0)
    mu = sums[:2 * _D] / _E
    var = sums[2 * _D:] / _E - mu * mu
    g2 = jnp.concatenate([gamma, gamma])
    be2 = jnp.concatenate([beta, beta])
    a_bn = g2 / jnp.sqrt(var + 1e-5)
    b_bn = be2 - mu * a_bn

    wblk = jnp.zeros((2 * _D, 2 * _D), jnp.float32)
    wblk = wblk.at[:_D, :_D].set(Wg2a).at[_D:, _D:].set(Wg2b)

    se = sep_subgraphs.reshape(-1, 2)
    s_se = jnp.concatenate([se[:, 0], se[:, 1]])
    d_se = jnp.concatenate([se[:, 1], se[:, 0]])
    idx2 = jnp.take(sub_graphs, s_se)
    deg2 = jax.ops.segment_sum(jnp.ones((_E,), jnp.float32), d_se,
                               num_segments=_E)
    agg = jax.ops.segment_sum(jnp.take(ef, idx2, axis=0), d_se,
                              num_segments=_E)
    sg = jnp.take(ef, sub_graphs, axis=0)

    wt = (W1a, b1a[None], W2a, b2a[None], W3a, b3a[None],
          W1b, b1b[None], W2b, b2b[None], W3b, b3b[None])
    return _mlp(sg, agg, deg2[:, None], a_bn[None], b_bn[None], wblk, wt)


# speedup vs baseline: 1.6584x; 1.6584x over previous
import functools

import jax
import jax.numpy as jnp
from jax import lax
from jax.experimental import pallas as pl
from jax.experimental.pallas import tpu as pltpu
from jax.experimental.pallas import tpu_sc as plsc

_N = 10000
_D = 128
_E = 160000
_S = 16
_H = 1024

_INTERPRET = False

_MESH = plsc.VectorSubcoreMesh(core_axis_name="c", subcore_axis_name="s")
_NC = 2
_NS = 16


def _lrelu(x):
    return jnp.where(x >= 0, x, 0.01 * x)


_A_CH = 256
_NPAD = 10240


_A_ROWS = 5632
_A_HALF = _N // 2


def _gcn1_agg_sc(x, src, dst, zeros_nd):
    per_tile = _E // _NS

    @functools.partial(
        pl.kernel,
        out_type=jax.ShapeDtypeStruct((_NC, _A_ROWS, _D), jnp.float32),
        mesh=_MESH,
        scratch_types=dict(
            acc=pltpu.VMEM_SHARED((_A_ROWS, _D), jnp.float32),
            gidx=pltpu.VMEM((_A_CH,), jnp.int32),
            sidx=pltpu.VMEM((_A_CH,), jnp.int32),
            rows=pltpu.VMEM((_A_CH, _D), jnp.float32),
            gidx_t=pltpu.VMEM((16,), jnp.int32),
            sidx_t=pltpu.VMEM((16,), jnp.int32),
            rows_t=pltpu.VMEM((16, _D), jnp.float32),
            sem=pltpu.SemaphoreType.DMA,
        ),
    )
    def k(x_hbm, src_hbm, dst_hbm, z_nd, a1_out, *, acc,
          gidx, sidx, rows, gidx_t, sidx_t, rows_t, sem):
        core = lax.axis_index("c")
        sid = lax.axis_index("s")
        stripe = _A_ROWS // _NS
        r0 = sid * stripe
        pltpu.sync_copy(z_nd.at[pl.ds(r0, stripe)], acc.at[pl.ds(r0, stripe)])
        plsc.subcore_barrier()
        base = sid * per_tile
        nb = core * _A_HALF

        def clamp(buf, n):
            @pl.loop(0, n // 16)
            def _(i):
                iv = buf[pl.ds(i * 16, 16)]
                lv = iv - nb
                m = (lv >= 0) & (lv < _A_HALF)
                buf[pl.ds(i * 16, 16)] = jnp.where(m, lv, _A_HALF)

        def chunk(off, gref, sref, gb, sb, rb, n):
            pltpu.sync_copy(gref.at[pl.ds(off, n)], gb)
            pltpu.sync_copy(sref.at[pl.ds(off, n)], sb)
            clamp(sb, n)
            pltpu.async_copy(x_hbm.at[gb], rb, sem).wait()
            pltpu.sync_copy(rb, acc.at[sb], add=True)

        nfull = per_tile // _A_CH
        tail = per_tile - nfull * _A_CH

        def direction(gref, sref):
            @pl.loop(0, nfull)
            def _(c):
                chunk(base + c * _A_CH, gref, sref, gidx, sidx, rows, _A_CH)
            if tail:
                chunk(base + nfull * _A_CH, gref, sref, gidx_t, sidx_t,
                      rows_t, tail)

        direction(src_hbm, dst_hbm)
        direction(dst_hbm, src_hbm)

        plsc.subcore_barrier()
        pltpu.sync_copy(acc.at[pl.ds(r0, stripe)],
                        a1_out.at[core, pl.ds(r0, stripe)])

    return k(x, src, dst, zeros_nd)


_C_CH = 128


def _edge_features_sc(h, src, dst, actions):
    per_tile = _E // (_NC * _NS)
    nfull = per_tile // _C_CH
    tail = per_tile - nfull * _C_CH
    w2 = 2 * _D

    @functools.partial(
        pl.kernel,
        out_type=jax.ShapeDtypeStruct((_E, w2), jnp.float32),
        mesh=_MESH,
        scratch_types=dict(
            hs=pltpu.VMEM((_C_CH, w2), jnp.float32),
            hd=pltpu.VMEM((_C_CH, w2), jnp.float32),
            sidx=pltpu.VMEM((_C_CH,), jnp.int32),
            didx=pltpu.VMEM((_C_CH,), jnp.int32),
            act=pltpu.VMEM((_C_CH,), jnp.float32),
            hs_t=pltpu.VMEM((8, w2), jnp.float32),
            hd_t=pltpu.VMEM((8, w2), jnp.float32),
            sidx_t=pltpu.VMEM((8,), jnp.int32),
            didx_t=pltpu.VMEM((8,), jnp.int32),
            act_t=pltpu.VMEM((16,), jnp.float32),
            sem=pltpu.SemaphoreType.DMA,
            sem2=pltpu.SemaphoreType.DMA,
        ),
    )
    def k(h_hbm, src_hbm, dst_hbm, act_hbm, ef_out, *, hs, hd, sidx,
          didx, act, hs_t, hd_t, sidx_t, didx_t, act_t, sem, sem2):
        core = lax.axis_index("c")
        sid = lax.axis_index("s")
        wid = core * _NS + sid
        base = wid * per_tile

        def rows16(hsb, hdb, row0, avec, lanes):
            for l in range(lanes):
                r = row0 + l
                av = avec[l]
                for j in range(w2 // 16):
                    cl = pl.ds(j * 16, 16)
                    v = (hsb[r, cl] + hdb[r, cl]) * av
                    hsb[r, cl] = jnp.where(v >= 0, v, 0.01 * v)

        def chunk(off, sb, db, ab, hsb, hdb, n):
            pltpu.sync_copy(src_hbm.at[pl.ds(off, n)], sb)
            pltpu.sync_copy(dst_hbm.at[pl.ds(off, n)], db)
            pltpu.sync_copy(act_hbm.at[pl.ds(off, n)], ab.at[pl.ds(0, n)])
            cs = pltpu.async_copy(h_hbm.at[sb], hsb.at[pl.ds(0, n)], sem)
            cd = pltpu.async_copy(h_hbm.at[db], hdb.at[pl.ds(0, n)], sem2)
            cs.wait()
            cd.wait()

            @pl.loop(0, n // 16)
            def _(g):
                avec = ab[pl.ds(g * 16, 16)]
                rows16(hsb, hdb, g * 16, avec, 16)

            rem = n % 16
            if rem:
                avec = ab[pl.ds(0, 16)]
                rows16(hsb, hdb, n - rem, avec, rem)

            pltpu.sync_copy(hsb.at[pl.ds(0, n)], ef_out.at[pl.ds(off, n)])

        @pl.loop(0, nfull)
        def _(c):
            chunk(base + c * _C_CH, sidx, didx, act, hs, hd, _C_CH)
        if tail:
            chunk(base + nfull * _C_CH, sidx_t, didx_t, act_t, hs_t, hd_t,
                  tail)

    return k(h, src, dst, actions)


def _h_body(x_ref, a_ref, deg_ref, w_ref, out_ref):
    deg = jnp.maximum(deg_ref[...], 1.0)
    x2 = x_ref[...] + a_ref[...] / deg
    out_ref[...] = _lrelu(jnp.dot(x2, w_ref[...], preferred_element_type=jnp.float32))


def _h_matmul(x, a1, deg1, w1ab):
    bm = 2000
    grid = (_N // bm,)
    return pl.pallas_call(
        _h_body,
        grid=grid,
        in_specs=[
            pl.BlockSpec((bm, _D), lambda i: (i, 0)),
            pl.BlockSpec((bm, _D), lambda i: (i, 0)),
            pl.BlockSpec((bm, 1), lambda i: (i, 0)),
            pl.BlockSpec((_D, 2 * _D), lambda i: (0, 0)),
        ],
        out_specs=pl.BlockSpec((bm, 2 * _D), lambda i: (i, 0)),
        out_shape=jax.ShapeDtypeStruct((_N, 2 * _D), jnp.float32),
        interpret=_INTERPRET,
    )(x, a1, deg1, w1ab)


def _stats_body(ef_ref, out_ref):
    i = pl.program_id(0)
    blk = ef_ref[...]
    v = jnp.concatenate([jnp.sum(blk, axis=0), jnp.sum(blk * blk, axis=0)])

    @pl.when(i == 0)
    def _():
        out_ref[...] = v[None]

    @pl.when(i != 0)
    def _():
        out_ref[...] = out_ref[...] + v[None]


def _bn_stats(ef):
    bm = 4000
    return pl.pallas_call(
        _stats_body,
        grid=(_E // bm,),
        in_specs=[pl.BlockSpec((bm, 2 * _D), lambda i: (i, 0))],
        out_specs=pl.BlockSpec((1, 4 * _D), lambda i: (0, 0)),
        out_shape=jax.ShapeDtypeStruct((1, 4 * _D), jnp.float32),
        interpret=_INTERPRET,
    )(ef)


def _mlp_body(sgz_ref, agg_ref, deg_ref, a_ref, b_ref, wblk_ref,
              w1a_ref, b1a_ref, w2a_ref, b2a_ref, w3a_ref, b3a_ref,
              w1b_ref, b1b_ref, w2b_ref, b2b_ref, w3b_ref, b3b_ref,
              out_ref):
    degr = deg_ref[...]
    deg = jnp.maximum(degr, 1.0)
    ind = jnp.where(degr > 0, 1.0, 0.0)
    pre = (a_ref[...] * (sgz_ref[...] + agg_ref[...] / deg)
           + b_ref[...] * (1.0 + ind))
    g = jnp.dot(pre, wblk_ref[...], preferred_element_type=jnp.float32)
    lr = _lrelu(_lrelu(g))
    gb = lr.shape[0] // _S
    pooled = jnp.mean(lr.reshape(gb, _S, 2 * _D), axis=1)
    p1 = pooled[:, :_D]
    p2 = pooled[:, _D:]

    def value(p, w1, b1, w2, b2, w3, b3):
        x = _lrelu(jnp.dot(p, w1[...], preferred_element_type=jnp.float32) + b1[...])
        x = _lrelu(jnp.dot(x, w2[...], preferred_element_type=jnp.float32) + b2[...])
        return jnp.dot(x, w3[...], preferred_element_type=jnp.float32) + b3[...]

    q1 = value(p1, w1a_ref, b1a_ref, w2a_ref, b2a_ref, w3a_ref, b3a_ref)
    q2 = value(p2, w1b_ref, b1b_ref, w2b_ref, b2b_ref, w3b_ref, b3b_ref)
    out_ref[...] = jnp.concatenate([q1, q2], axis=-1)


def _mlp(sgz, agg, deg2, a_bn, b_bn, wblk, wt):
    bm = 6400
    gb = bm // _S
    grid = (_E // bm,)
    full = lambda shape: pl.BlockSpec(shape, lambda i: tuple(0 for _ in shape))
    return pl.pallas_call(
        _mlp_body,
        grid=grid,
        in_specs=[
            pl.BlockSpec((bm, 2 * _D), lambda i: (i, 0)),
            pl.BlockSpec((bm, 2 * _D), lambda i: (i, 0)),
            pl.BlockSpec((bm, 1), lambda i: (i, 0)),
            full((1, 2 * _D)), full((1, 2 * _D)),
            full((2 * _D, 2 * _D)),
            full((_D, _H)), full((1, _H)), full((_H, _H)), full((1, _H)),
            full((_H, 1)), full((1, 1)),
            full((_D, _H)), full((1, _H)), full((_H, _H)), full((1, _H)),
            full((_H, 1)), full((1, 1)),
        ],
        out_specs=pl.BlockSpec((gb, 2), lambda i: (i, 0)),
        out_shape=jax.ShapeDtypeStruct((_N, 2), jnp.float32),
        interpret=_INTERPRET,
    )(sgz, agg, deg2, a_bn, b_bn, wblk, *wt)


def kernel(node_features, actions, edge_index, sub_graphs, sep_subgraphs,
           Wg1a, Wg1b, Wg2a, Wg2b, gamma, beta,
           W1a, b1a, W2a, b2a, W3a, b3a,
           W1b, b1b, W2b, b2b, W3b, b3b):
    x = node_features
    src = edge_index[0]
    dst = edge_index[1]
    src2 = jnp.concatenate([src, dst])
    dst2 = jnp.concatenate([dst, src])

    a1p = _gcn1_agg_sc(x, src, dst, jnp.zeros((_A_ROWS, _D), jnp.float32))
    a1 = jnp.concatenate([a1p[0, :_A_HALF], a1p[1, :_A_HALF]], axis=0)
    dst2 = jnp.concatenate([dst, src])
    deg1 = jax.ops.segment_sum(jnp.ones((2 * _E,), jnp.float32), dst2,
                               num_segments=_N)

    w1ab = jnp.concatenate([Wg1a, Wg1b], axis=1)
    h = _h_matmul(x, a1, deg1[:, None], w1ab)

    ef = _edge_features_sc(h, src, dst, actions)
    sums = _bn_stats(ef)[0]
    mu = sums[:2 * _D] / _E
    var = sums[2 * _D:] / _E - mu * mu
    g2 = jnp.concatenate([gamma, gamma])
    be2 = jnp.concatenate([beta, beta])
    a_bn = g2 / jnp.sqrt(var + 1e-5)
    b_bn = be2 - mu * a_bn

    wblk = jnp.zeros((2 * _D, 2 * _D), jnp.float32)
    wblk = wblk.at[:_D, :_D].set(Wg2a).at[_D:, _D:].set(Wg2b)

    se = sep_subgraphs.reshape(-1, 2)
    s_se = jnp.concatenate([se[:, 0], se[:, 1]])
    d_se = jnp.concatenate([se[:, 1], se[:, 0]])
    idx2 = jnp.take(sub_graphs, s_se)
    deg2 = jax.ops.segment_sum(jnp.ones((_E,), jnp.float32), d_se,
                               num_segments=_E)
    agg = jax.ops.segment_sum(jnp.take(ef, idx2, axis=0), d_se,
                              num_segments=_E)
    sg = jnp.take(ef, sub_graphs, axis=0)

    wt = (W1a, b1a[None], W2a, b2a[None], W3a, b3a[None],
          W1b, b1b[None], W2b, b2b[None], W3b, b3b[None])
    return _mlp(sg, agg, deg2[:, None], a_bn[None], b_bn[None], wblk, wt)


# speedup vs baseline: 1.6598x; 1.0009x over previous
import functools

import jax
import jax.numpy as jnp
from jax import lax
from jax.experimental import pallas as pl
from jax.experimental.pallas import tpu as pltpu
from jax.experimental.pallas import tpu_sc as plsc

_N = 10000
_D = 128
_E = 160000
_S = 16
_H = 1024

_INTERPRET = False

_MESH = plsc.VectorSubcoreMesh(core_axis_name="c", subcore_axis_name="s")
_NC = 2
_NS = 16


def _lrelu(x):
    return jnp.where(x >= 0, x, 0.01 * x)


_A_CH = 256
_A_ROWS = 5632
_A_HALF = _N // 2


def _gcn1_agg_sc(x, src, dst, zeros_nd):
    per_tile = _E // _NS

    @functools.partial(
        pl.kernel,
        out_type=jax.ShapeDtypeStruct((_NC, _A_ROWS, _D), jnp.float32),
        mesh=_MESH,
        scratch_types=dict(
            acc=pltpu.VMEM_SHARED((_A_ROWS, _D), jnp.float32),
            gidx=pltpu.VMEM((_A_CH,), jnp.int32),
            sidx=pltpu.VMEM((_A_CH,), jnp.int32),
            rows=pltpu.VMEM((_A_CH, _D), jnp.float32),
            gidx_t=pltpu.VMEM((16,), jnp.int32),
            sidx_t=pltpu.VMEM((16,), jnp.int32),
            rows_t=pltpu.VMEM((16, _D), jnp.float32),
            sem=pltpu.SemaphoreType.DMA,
        ),
    )
    def k(x_hbm, src_hbm, dst_hbm, z_nd, a1_out, *, acc,
          gidx, sidx, rows, gidx_t, sidx_t, rows_t, sem):
        core = lax.axis_index("c")
        sid = lax.axis_index("s")
        stripe = _A_ROWS // _NS
        r0 = sid * stripe
        pltpu.sync_copy(z_nd.at[pl.ds(r0, stripe)], acc.at[pl.ds(r0, stripe)])
        plsc.subcore_barrier()
        base = sid * per_tile
        nb = core * _A_HALF

        def clamp(buf, n):
            @pl.loop(0, n // 16)
            def _(i):
                iv = buf[pl.ds(i * 16, 16)]
                lv = iv - nb
                m = (lv >= 0) & (lv < _A_HALF)
                buf[pl.ds(i * 16, 16)] = jnp.where(m, lv, _A_HALF)

        def chunk(off, gref, sref, gb, sb, rb, n):
            pltpu.sync_copy(gref.at[pl.ds(off, n)], gb)
            pltpu.sync_copy(sref.at[pl.ds(off, n)], sb)
            clamp(sb, n)
            pltpu.async_copy(x_hbm.at[gb], rb, sem).wait()
            pltpu.sync_copy(rb, acc.at[sb], add=True)

        nfull = per_tile // _A_CH
        tail = per_tile - nfull * _A_CH

        def direction(gref, sref):
            @pl.loop(0, nfull)
            def _(c):
                chunk(base + c * _A_CH, gref, sref, gidx, sidx, rows, _A_CH)
            if tail:
                chunk(base + nfull * _A_CH, gref, sref, gidx_t, sidx_t,
                      rows_t, tail)

        direction(src_hbm, dst_hbm)
        direction(dst_hbm, src_hbm)

        plsc.subcore_barrier()
        pltpu.sync_copy(acc.at[pl.ds(r0, stripe)],
                        a1_out.at[core, pl.ds(r0, stripe)])

    return k(x, src, dst, zeros_nd)


_C_CH = 128


def _edge_features_sc(h, src, dst, actions):
    per_tile = _E // (_NC * _NS)
    nfull = per_tile // _C_CH
    tail = per_tile - nfull * _C_CH
    w2 = 2 * _D

    @functools.partial(
        pl.kernel,
        out_type=jax.ShapeDtypeStruct((_E, w2), jnp.float32),
        mesh=_MESH,
        scratch_types=dict(
            hs=pltpu.VMEM((_C_CH, w2), jnp.float32),
            hd=pltpu.VMEM((_C_CH, w2), jnp.float32),
            sidx=pltpu.VMEM((_C_CH,), jnp.int32),
            didx=pltpu.VMEM((_C_CH,), jnp.int32),
            act=pltpu.VMEM((_C_CH,), jnp.float32),
            hs_t=pltpu.VMEM((8, w2), jnp.float32),
            hd_t=pltpu.VMEM((8, w2), jnp.float32),
            sidx_t=pltpu.VMEM((8,), jnp.int32),
            didx_t=pltpu.VMEM((8,), jnp.int32),
            act_t=pltpu.VMEM((16,), jnp.float32),
            sem=pltpu.SemaphoreType.DMA,
            sem2=pltpu.SemaphoreType.DMA,
        ),
    )
    def k(h_hbm, src_hbm, dst_hbm, act_hbm, ef_out, *, hs, hd, sidx,
          didx, act, hs_t, hd_t, sidx_t, didx_t, act_t, sem, sem2):
        core = lax.axis_index("c")
        sid = lax.axis_index("s")
        wid = core * _NS + sid
        base = wid * per_tile

        def rows16(hsb, hdb, row0, avec, lanes):
            for l in range(lanes):
                r = row0 + l
                av = avec[l]
                for j in range(w2 // 16):
                    cl = pl.ds(j * 16, 16)
                    v = (hsb[r, cl] + hdb[r, cl]) * av
                    hsb[r, cl] = jnp.where(v >= 0, v, 0.01 * v)

        def chunk(off, sb, db, ab, hsb, hdb, n):
            pltpu.sync_copy(src_hbm.at[pl.ds(off, n)], sb)
            pltpu.sync_copy(dst_hbm.at[pl.ds(off, n)], db)
            pltpu.sync_copy(act_hbm.at[pl.ds(off, n)], ab.at[pl.ds(0, n)])
            cs = pltpu.async_copy(h_hbm.at[sb], hsb.at[pl.ds(0, n)], sem)
            cd = pltpu.async_copy(h_hbm.at[db], hdb.at[pl.ds(0, n)], sem2)
            cs.wait()
            cd.wait()

            @pl.loop(0, n // 16)
            def _(g):
                avec = ab[pl.ds(g * 16, 16)]
                rows16(hsb, hdb, g * 16, avec, 16)

            rem = n % 16
            if rem:
                avec = ab[pl.ds(0, 16)]
                rows16(hsb, hdb, n - rem, avec, rem)

            pltpu.sync_copy(hsb.at[pl.ds(0, n)], ef_out.at[pl.ds(off, n)])

        @pl.loop(0, nfull)
        def _(c):
            chunk(base + c * _C_CH, sidx, didx, act, hs, hd, _C_CH)
        if tail:
            chunk(base + nfull * _C_CH, sidx_t, didx_t, act_t, hs_t, hd_t,
                  tail)

    return k(h, src, dst, actions)


def _h_body(x_ref, a_ref, deg_ref, w_ref, out_ref):
    deg = jnp.maximum(deg_ref[...], 1.0)
    x2 = x_ref[...] + a_ref[...] / deg
    out_ref[...] = _lrelu(jnp.dot(x2, w_ref[...], preferred_element_type=jnp.float32))


def _h_matmul(x, a1, deg1, w1ab):
    bm = 2000
    grid = (_N // bm,)
    return pl.pallas_call(
        _h_body,
        grid=grid,
        in_specs=[
            pl.BlockSpec((bm, _D), lambda i: (i, 0)),
            pl.BlockSpec((bm, _D), lambda i: (i, 0)),
            pl.BlockSpec((bm, 1), lambda i: (i, 0)),
            pl.BlockSpec((_D, 2 * _D), lambda i: (0, 0)),
        ],
        out_specs=pl.BlockSpec((bm, 2 * _D), lambda i: (i, 0)),
        out_shape=jax.ShapeDtypeStruct((_N, 2 * _D), jnp.float32),
        interpret=_INTERPRET,
    )(x, a1, deg1, w1ab)


def _stats_body(ef_ref, out_ref):
    i = pl.program_id(0)
    blk = ef_ref[...]
    v = jnp.concatenate([jnp.sum(blk, axis=0), jnp.sum(blk * blk, axis=0)])

    @pl.when(i == 0)
    def _():
        out_ref[...] = v[None]

    @pl.when(i != 0)
    def _():
        out_ref[...] = out_ref[...] + v[None]


def _bn_stats(ef):
    bm = 4000
    return pl.pallas_call(
        _stats_body,
        grid=(_E // bm,),
        in_specs=[pl.BlockSpec((bm, 2 * _D), lambda i: (i, 0))],
        out_specs=pl.BlockSpec((1, 4 * _D), lambda i: (0, 0)),
        out_shape=jax.ShapeDtypeStruct((1, 4 * _D), jnp.float32),
        interpret=_INTERPRET,
    )(ef)


def _mlp_body(sgz_ref, agg_ref, deg_ref, a_ref, b_ref, wblk_ref,
              w1a_ref, b1a_ref, w2a_ref, b2a_ref, w3a_ref, b3a_ref,
              w1b_ref, b1b_ref, w2b_ref, b2b_ref, w3b_ref, b3b_ref,
              out_ref):
    degr = deg_ref[...]
    deg = jnp.maximum(degr, 1.0)
    ind = jnp.where(degr > 0, 1.0, 0.0)
    pre = (a_ref[...] * (sgz_ref[...] + agg_ref[...] / deg)
           + b_ref[...] * (1.0 + ind))
    g = jnp.dot(pre, wblk_ref[...], preferred_element_type=jnp.float32)
    lr = _lrelu(_lrelu(g))
    gb = lr.shape[0] // _S
    pooled = jnp.mean(lr.reshape(gb, _S, 2 * _D), axis=1)
    p1 = pooled[:, :_D]
    p2 = pooled[:, _D:]

    def value(p, w1, b1, w2, b2, w3, b3):
        x = _lrelu(jnp.dot(p, w1[...], preferred_element_type=jnp.float32) + b1[...])
        x = _lrelu(jnp.dot(x, w2[...], preferred_element_type=jnp.float32) + b2[...])
        return jnp.dot(x, w3[...], preferred_element_type=jnp.float32) + b3[...]

    q1 = value(p1, w1a_ref, b1a_ref, w2a_ref, b2a_ref, w3a_ref, b3a_ref)
    q2 = value(p2, w1b_ref, b1b_ref, w2b_ref, b2b_ref, w3b_ref, b3b_ref)
    out_ref[...] = jnp.concatenate([q1, q2], axis=-1)


def _mlp(sgz, agg, deg2, a_bn, b_bn, wblk, wt):
    bm = 6400
    gb = bm // _S
    grid = (_E // bm,)
    full = lambda shape: pl.BlockSpec(shape, lambda i: tuple(0 for _ in shape))
    return pl.pallas_call(
        _mlp_body,
        grid=grid,
        in_specs=[
            pl.BlockSpec((bm, 2 * _D), lambda i: (i, 0)),
            pl.BlockSpec((bm, 2 * _D), lambda i: (i, 0)),
            pl.BlockSpec((bm, 1), lambda i: (i, 0)),
            full((1, 2 * _D)), full((1, 2 * _D)),
            full((2 * _D, 2 * _D)),
            full((_D, _H)), full((1, _H)), full((_H, _H)), full((1, _H)),
            full((_H, 1)), full((1, 1)),
            full((_D, _H)), full((1, _H)), full((_H, _H)), full((1, _H)),
            full((_H, 1)), full((1, 1)),
        ],
        out_specs=pl.BlockSpec((gb, 2), lambda i: (i, 0)),
        out_shape=jax.ShapeDtypeStruct((_N, 2), jnp.float32),
        interpret=_INTERPRET,
    )(sgz, agg, deg2, a_bn, b_bn, wblk, *wt)


def kernel(node_features, actions, edge_index, sub_graphs, sep_subgraphs,
           Wg1a, Wg1b, Wg2a, Wg2b, gamma, beta,
           W1a, b1a, W2a, b2a, W3a, b3a,
           W1b, b1b, W2b, b2b, W3b, b3b):
    x = node_features
    src = edge_index[0]
    dst = edge_index[1]
    src2 = jnp.concatenate([src, dst])
    dst2 = jnp.concatenate([dst, src])

    a1p = _gcn1_agg_sc(x, src, dst, jnp.zeros((_A_ROWS, _D), jnp.float32))
    a1 = jnp.concatenate([a1p[0, :_A_HALF], a1p[1, :_A_HALF]], axis=0)
    dst2 = jnp.concatenate([dst, src])
    deg1 = jax.ops.segment_sum(jnp.ones((2 * _E,), jnp.float32), dst2,
                               num_segments=_N)

    w1ab = jnp.concatenate([Wg1a, Wg1b], axis=1)
    h = _h_matmul(x, a1, deg1[:, None], w1ab)

    ef = _edge_features_sc(h, src, dst, actions)
    sums = _bn_stats(ef)[0]
    mu = sums[:2 * _D] / _E
    var = sums[2 * _D:] / _E - mu * mu
    g2 = jnp.concatenate([gamma, gamma])
    be2 = jnp.concatenate([beta, beta])
    a_bn = g2 / jnp.sqrt(var + 1e-5)
    b_bn = be2 - mu * a_bn

    wblk = jnp.zeros((2 * _D, 2 * _D), jnp.float32)
    wblk = wblk.at[:_D, :_D].set(Wg2a).at[_D:, _D:].set(Wg2b)

    se = sep_subgraphs.reshape(-1, 2)
    s_se = jnp.concatenate([se[:, 0], se[:, 1]])
    d_se = jnp.concatenate([se[:, 1], se[:, 0]])
    idx2 = jnp.take(sub_graphs, s_se)
    deg2 = jax.ops.segment_sum(jnp.ones((_E,), jnp.float32), d_se,
                               num_segments=_E)
    agg = jax.ops.segment_sum(jnp.take(ef, idx2, axis=0), d_se,
                              num_segments=_E)
    sg = jnp.take(ef, sub_graphs, axis=0)

    wt = (W1a, b1a[None], W2a, b2a[None], W3a, b3a[None],
          W1b, b1b[None], W2b, b2b[None], W3b, b3b[None])
    return _mlp(sg, agg, deg2[:, None], a_bn[None], b_bn[None], wblk, wt)
